# baseline (speedup 1.0000x reference)
import functools
import math

import jax
import jax.numpy as jnp
from jax import lax
from jax.experimental import pallas as pl
from jax.experimental.pallas import tpu as pltpu
from jax.experimental.pallas import tpu_sc as plsc

N = 10000
E = 320000
D_IN = 128
D_HID = 128
D_OUT = 64
N_CLS = 16

NW = 32
CHUNK = 128
NBUF = 2
NCH = 80
NPH = 2
IDXH = NCH // NPH
P = NCH * CHUNK
E_PAD = NW * P
RPT = 632
N_ACC = RPT * 16

_mesh = plsc.VectorSubcoreMesh(core_axis_name="c", subcore_axis_name="s")


_PIECES = []
_off = 0
while _off < RPT:
    _sz = min(CHUNK, RPT - _off)
    _PIECES.append((_off, _sz))
    _off += _sz


def _agg_body(D, y, srcp, dstp, zD, part, acc, src_v, dst_v, *bufsem):
    bufs = list(bufsem[:NBUF])
    gs = list(bufsem[NBUF:2 * NBUF])
    ss = list(bufsem[2 * NBUF:])
    b0, g0 = bufs[0], gs[0]
    c = lax.axis_index("c")
    s = lax.axis_index("s")
    wid = s * 2 + c
    r0 = s * RPT
    pltpu.sync_copy(zD, b0)
    hs = [pltpu.async_copy(b0.at[pl.ds(0, sz)], acc.at[pl.ds(r0 + off, sz)], g0)
          for off, sz in _PIECES]
    for h in hs:
        h.wait()
    plsc.subcore_barrier()

    def fire_gather(j, b):
        pltpu.async_copy(y.at[src_v.at[j]], bufs[b], gs[b])

    def wait_gather(b):
        pltpu.make_async_copy(y.at[src_v.at[0]], bufs[b], gs[b]).wait()

    def fire_scatter(j, b):
        pltpu.async_copy(bufs[b], acc.at[dst_v.at[j]], ss[b], add=True)

    def wait_scatter(b):
        pltpu.make_async_copy(bufs[b], acc.at[dst_v.at[0]], ss[b]).wait()

    for ph in range(NPH):
        row0 = wid * NCH + ph * IDXH
        pltpu.sync_copy(srcp.at[pl.ds(row0, IDXH)], src_v)
        pltpu.sync_copy(dstp.at[pl.ds(row0, IDXH)], dst_v)

        def body(j, carry):
            pltpu.async_copy(y.at[src_v.at[j]], bufs[0], gs[0]).wait()
            pltpu.sync_copy(bufs[0], acc.at[dst_v.at[j]], add=True)
            return carry

        lax.fori_loop(0, IDXH, body, 0)
    plsc.subcore_barrier()
    stores = {}
    for i, (off, sz) in enumerate(_PIECES):
        b = i % NBUF
        if b in stores:
            stores[b].wait()
        pltpu.async_copy(acc.at[pl.ds(r0 + off, sz)], bufs[b].at[pl.ds(0, sz)],
                         gs[b]).wait()
        stores[b] = pltpu.async_copy(bufs[b].at[pl.ds(0, sz)],
                                     part.at[c, pl.ds(r0 + off, sz)], ss[b])
    for h in stores.values():
        h.wait()


def _make_agg(D):
    return pl.kernel(
        functools.partial(_agg_body, D),
        mesh=_mesh,
        out_type=jax.ShapeDtypeStruct((2, N_ACC, D), jnp.float32),
        scratch_types=[
            pltpu.VMEM_SHARED((N_ACC, D), jnp.float32),
            pltpu.VMEM((IDXH, CHUNK), jnp.int32),
            pltpu.VMEM((IDXH, CHUNK), jnp.int32),
        ] + [pltpu.VMEM((CHUNK, D), jnp.float32)] * NBUF
          + [pltpu.SemaphoreType.DMA] * (2 * NBUF),
        compiler_params=pltpu.CompilerParams(use_tc_tiling_on_sc=(D == 128)),
    )


def _cnt_body(dstp, z16, ones, cntout, cacc, dst_v, ones_v, sem):
    c = lax.axis_index("c")
    s = lax.axis_index("s")
    wid = s * 2 + c
    r0 = s * RPT
    pltpu.sync_copy(dstp.at[pl.ds(wid * NCH, NCH)], dst_v)
    pltpu.sync_copy(z16, ones_v)
    hs = [pltpu.async_copy(ones_v.at[pl.ds(0, sz)], cacc.at[pl.ds(r0 + off, sz)], sem)
          for off, sz in _PIECES]
    for h in hs:
        h.wait()
    pltpu.sync_copy(ones, ones_v)
    plsc.subcore_barrier()

    def step(k, carry):
        for b in range(NBUF):
            pltpu.async_copy(ones_v, cacc.at[dst_v.at[k * NBUF + b]], sem,
                             add=True)
        for b in range(NBUF):
            pltpu.make_async_copy(ones_v, cacc.at[dst_v.at[k * NBUF + b]],
                                  sem).wait()
        return carry

    lax.fori_loop(0, NCH // NBUF, step, 0)
    plsc.subcore_barrier()
    for off, sz in _PIECES:
        pltpu.sync_copy(cacc.at[pl.ds(r0 + off, sz)], ones_v.at[pl.ds(0, sz)])
        pltpu.sync_copy(ones_v.at[pl.ds(0, sz)], cntout.at[c, pl.ds(r0 + off, sz)])


_cnt = pl.kernel(
    _cnt_body,
    mesh=_mesh,
    out_type=jax.ShapeDtypeStruct((2, N_ACC, 16), jnp.float32),
    scratch_types=[
        pltpu.VMEM_SHARED((N_ACC, 16), jnp.float32),
        pltpu.VMEM((NCH, CHUNK), jnp.int32),
        pltpu.VMEM((CHUNK, 16), jnp.float32),
        pltpu.SemaphoreType.DMA,
    ],
    compiler_params=pltpu.CompilerParams(use_tc_tiling_on_sc=False),
)

_agg128 = _make_agg(D_HID)
_agg64 = _make_agg(D_OUT)


_SQRT2 = math.sqrt(2.0)


def _gelu(h):
    return 0.5 * h * (1.0 + lax.erf(h / _SQRT2))


def _lnorm(h, g, b):
    m = jnp.mean(h, axis=1, keepdims=True)
    v = jnp.mean((h - m) * (h - m), axis=1, keepdims=True)
    return (h - m) * lax.rsqrt(v + 1e-5) * g + b


BN = 1000


def _tc12_body(p0, p1, c0, c1, x, Wl, bl, Wr, g, be, o):
    cnt = jnp.maximum(c0[:, :1] + c1[:, :1], 1.0)
    mean = (p0[...] + p1[...]) / cnt
    h = (lax.dot_general(mean, Wl[...], (((1,), (1,)), ((), ())),
                         preferred_element_type=jnp.float32)
         + lax.dot_general(x[...], Wr[...], (((1,), (1,)), ((), ())),
                           preferred_element_type=jnp.float32)
         + bl[...])
    h = _gelu(_lnorm(h, g[...], be[...]))
    o[...] = h + x[...]


def _tc2_body(p0, p1, c0, c1, x, Wl, bl, Wr, g, be, Wol, o, oy):
    cnt = jnp.maximum(c0[:, :1] + c1[:, :1], 1.0)
    mean = (p0[...] + p1[...]) / cnt
    h = (lax.dot_general(mean, Wl[...], (((1,), (1,)), ((), ())),
                         preferred_element_type=jnp.float32)
         + lax.dot_general(x[...], Wr[...], (((1,), (1,)), ((), ())),
                           preferred_element_type=jnp.float32)
         + bl[...])
    h = _gelu(_lnorm(h, g[...], be[...]))
    x2 = h + x[...]
    o[...] = x2
    oy[...] = lax.dot_general(x2, Wol[...], (((1,), (1,)), ((), ())),
                              preferred_element_type=jnp.float32)


def _tc3_body(p0, p1, c0, c1, x2, Wor, bol, gc, bec, Wc, bc, o):
    cnt = jnp.maximum(c0[:, :1] + c1[:, :1], 1.0)
    mean = (p0[...] + p1[...]) / cnt
    h = (mean + bol[...]
         + lax.dot_general(x2[...], Wor[...], (((1,), (1,)), ((), ())),
                           preferred_element_type=jnp.float32))
    h = _lnorm(_gelu(h), gc[...], bec[...])
    o[...] = (lax.dot_general(h, Wc[...], (((1,), (1,)), ((), ())),
                              preferred_element_type=jnp.float32)
              + bc[...])


def _bspec(shape):
    nd = len(shape)
    return pl.BlockSpec(shape, lambda i: (i,) + (0,) * (nd - 1))


def _wspec(shape):
    nd = len(shape)
    return pl.BlockSpec(shape, lambda i: (0,) * nd)


def _tc_call(body, n_node_in, w_shapes, out_shapes):
    grid = N // BN
    in_specs = ([_bspec((BN, s)) for s in n_node_in]
                + [_wspec(s) for s in w_shapes])
    out_specs = [_bspec((BN, s)) for s in out_shapes]
    out_shape = [jax.ShapeDtypeStruct((N, s), jnp.float32) for s in out_shapes]
    if len(out_shapes) == 1:
        out_specs, out_shape = out_specs[0], out_shape[0]
    return pl.pallas_call(
        body, grid=(grid,), in_specs=in_specs,
        out_specs=out_specs, out_shape=out_shape)


_tc1 = _tc_call(_tc12_body, [D_HID, D_HID, 16, 16, D_HID],
                [(D_HID, D_HID), (1, D_HID), (D_HID, D_HID), (1, D_HID), (1, D_HID)],
                [D_HID])
_tc2 = _tc_call(_tc2_body, [D_HID, D_HID, 16, 16, D_HID],
                [(D_HID, D_HID), (1, D_HID), (D_HID, D_HID), (1, D_HID), (1, D_HID),
                 (D_OUT, D_HID)],
                [D_HID, D_OUT])
_tc3 = _tc_call(_tc3_body, [D_OUT, D_OUT, 16, 16, D_HID],
                [(D_OUT, D_HID), (1, D_OUT), (1, D_OUT), (1, D_OUT),
                 (N_CLS, D_OUT), (1, N_CLS)],
                [N_CLS])


def kernel(x, edge_index, W1l, b1l, W1r, g1, be1, W2l, b2l, W2r, g2, be2,
           Wol, bol, Wor, gc, bec, Wc, bc):
    src = edge_index[0]
    dst = edge_index[1]
    pad = E_PAD - E
    srcp = jnp.concatenate([src, jnp.zeros((pad,), jnp.int32)]).reshape(
        NW * NCH, CHUNK)
    dstp = jnp.concatenate([dst, jnp.full((pad,), N, jnp.int32)]).reshape(
        NW * NCH, CHUNK)
    z128 = jnp.zeros((CHUNK, D_HID), jnp.float32)
    z64 = jnp.zeros((CHUNK, D_OUT), jnp.float32)
    z16 = jnp.zeros((CHUNK, 16), jnp.float32)
    ones = jnp.ones((CHUNK, 16), jnp.float32)
    r = lambda v: v.reshape(1, -1)

    cntp = _cnt(dstp, z16, ones)
    part1 = _agg128(x, srcp, dstp, z128)
    c0, c1 = cntp[0], cntp[1]
    x1 = _tc1(part1[0], part1[1], c0, c1, x,
              W1l, r(b1l), W1r, r(g1), r(be1))
    part2 = _agg128(x1, srcp, dstp, z128)
    x2, y3 = _tc2(part2[0], part2[1], c0, c1, x1,
                  W2l, r(b2l), W2r, r(g2), r(be2), Wol)
    part3 = _agg64(y3, srcp, dstp, z64)
    return _tc3(part3[0], part3[1], c0, c1, x2,
                Wor, r(bol), r(gc), r(bec), Wc, r(bc))


# speedup vs baseline: 1.1921x; 1.1921x over previous
import functools
import math

import jax
import jax.numpy as jnp
from jax import lax
from jax.experimental import pallas as pl
from jax.experimental.pallas import tpu as pltpu
from jax.experimental.pallas import tpu_sc as plsc

N = 10000
E = 320000
D_IN = 128
D_HID = 128
D_OUT = 64
N_CLS = 16

NW = 32
CHUNK = 128
NCH = 79
P = NCH * CHUNK
E_PAD = NW * P
RPT = 632
N_ACC = RPT * 16

_mesh = plsc.VectorSubcoreMesh(core_axis_name="c", subcore_axis_name="s")


_PIECES = []
_off = 0
while _off < RPT:
    _sz = min(CHUNK, RPT - _off)
    _PIECES.append((_off, _sz))
    _off += _sz


def _agg_body(D, y, srcp, dstp, zD, part, acc, src_v, dst_v, rows_v, sem, sem2):
    c = lax.axis_index("c")
    s = lax.axis_index("s")
    wid = s * 2 + c
    r0 = s * RPT
    pltpu.sync_copy(zD, rows_v)
    hs = [pltpu.async_copy(rows_v.at[pl.ds(0, sz)], acc.at[pl.ds(r0 + off, sz)],
                           sem)
          for off, sz in _PIECES]
    for h in hs:
        h.wait()
    plsc.subcore_barrier()

    def step(j, carry):
        base = wid * P + j * CHUNK
        pltpu.sync_copy(srcp.at[pl.ds(base, CHUNK)], src_v)
        pltpu.sync_copy(dstp.at[pl.ds(base, CHUNK)], dst_v)
        pltpu.async_copy(y.at[src_v], rows_v, sem).wait()
        pltpu.sync_copy(rows_v, acc.at[dst_v], add=True)
        return carry

    lax.fori_loop(0, NCH, step, 0)
    plsc.subcore_barrier()
    prev = None
    for off, sz in _PIECES:
        if prev is not None:
            prev.wait()
        pltpu.async_copy(acc.at[pl.ds(r0 + off, sz)], rows_v.at[pl.ds(0, sz)],
                         sem).wait()
        prev = pltpu.async_copy(rows_v.at[pl.ds(0, sz)],
                                part.at[c, pl.ds(r0 + off, sz)], sem2)
    prev.wait()


def _make_agg(D):
    return pl.kernel(
        functools.partial(_agg_body, D),
        mesh=_mesh,
        out_type=jax.ShapeDtypeStruct((2, N_ACC, D), jnp.float32),
        scratch_types=[
            pltpu.VMEM_SHARED((N_ACC, D), jnp.float32),
            pltpu.VMEM((CHUNK,), jnp.int32),
            pltpu.VMEM((CHUNK,), jnp.int32),
            pltpu.VMEM((CHUNK, D), jnp.float32),
            pltpu.SemaphoreType.DMA,
            pltpu.SemaphoreType.DMA,
        ],
        compiler_params=pltpu.CompilerParams(use_tc_tiling_on_sc=(D == 128)),
    )


def _cnt_body(dstp, z16, ones, cntout, cacc, dst_v, ones_v, sem):
    c = lax.axis_index("c")
    s = lax.axis_index("s")
    wid = s * 2 + c
    r0 = s * RPT
    pltpu.sync_copy(z16, ones_v)
    hs = [pltpu.async_copy(ones_v.at[pl.ds(0, sz)], cacc.at[pl.ds(r0 + off, sz)], sem)
          for off, sz in _PIECES]
    for h in hs:
        h.wait()
    pltpu.sync_copy(ones, ones_v)
    plsc.subcore_barrier()

    def step(j, carry):
        base = wid * P + j * CHUNK
        pltpu.sync_copy(dstp.at[pl.ds(base, CHUNK)], dst_v)
        pltpu.sync_copy(ones_v, cacc.at[dst_v], add=True)
        return carry

    lax.fori_loop(0, NCH, step, 0)
    plsc.subcore_barrier()
    for off, sz in _PIECES:
        pltpu.sync_copy(cacc.at[pl.ds(r0 + off, sz)], ones_v.at[pl.ds(0, sz)])
        pltpu.sync_copy(ones_v.at[pl.ds(0, sz)], cntout.at[c, pl.ds(r0 + off, sz)])


_cnt = pl.kernel(
    _cnt_body,
    mesh=_mesh,
    out_type=jax.ShapeDtypeStruct((2, N_ACC, 16), jnp.float32),
    scratch_types=[
        pltpu.VMEM_SHARED((N_ACC, 16), jnp.float32),
        pltpu.VMEM((CHUNK,), jnp.int32),
        pltpu.VMEM((CHUNK, 16), jnp.float32),
        pltpu.SemaphoreType.DMA,
    ],
    compiler_params=pltpu.CompilerParams(use_tc_tiling_on_sc=False),
)

_agg128 = _make_agg(D_HID)
_agg64 = _make_agg(D_OUT)


_SQRT2 = math.sqrt(2.0)


def _gelu(h):
    return 0.5 * h * (1.0 + lax.erf(h / _SQRT2))


def _lnorm(h, g, b):
    m = jnp.mean(h, axis=1, keepdims=True)
    v = jnp.mean((h - m) * (h - m), axis=1, keepdims=True)
    return (h - m) * lax.rsqrt(v + 1e-5) * g + b


BN = 1000


def _tc12_body(p0, p1, c0, c1, x, Wl, bl, Wr, g, be, o):
    cnt = jnp.maximum(c0[:, :1] + c1[:, :1], 1.0)
    mean = (p0[...] + p1[...]) / cnt
    h = (lax.dot_general(mean, Wl[...], (((1,), (1,)), ((), ())),
                         preferred_element_type=jnp.float32)
         + lax.dot_general(x[...], Wr[...], (((1,), (1,)), ((), ())),
                           preferred_element_type=jnp.float32)
         + bl[...])
    h = _gelu(_lnorm(h, g[...], be[...]))
    o[...] = h + x[...]


def _tc2_body(p0, p1, c0, c1, x, Wl, bl, Wr, g, be, Wol, o, oy):
    cnt = jnp.maximum(c0[:, :1] + c1[:, :1], 1.0)
    mean = (p0[...] + p1[...]) / cnt
    h = (lax.dot_general(mean, Wl[...], (((1,), (1,)), ((), ())),
                         preferred_element_type=jnp.float32)
         + lax.dot_general(x[...], Wr[...], (((1,), (1,)), ((), ())),
                           preferred_element_type=jnp.float32)
         + bl[...])
    h = _gelu(_lnorm(h, g[...], be[...]))
    x2 = h + x[...]
    o[...] = x2
    oy[...] = lax.dot_general(x2, Wol[...], (((1,), (1,)), ((), ())),
                              preferred_element_type=jnp.float32)


def _tc3_body(p0, p1, c0, c1, x2, Wor, bol, gc, bec, Wc, bc, o):
    cnt = jnp.maximum(c0[:, :1] + c1[:, :1], 1.0)
    mean = (p0[...] + p1[...]) / cnt
    h = (mean + bol[...]
         + lax.dot_general(x2[...], Wor[...], (((1,), (1,)), ((), ())),
                           preferred_element_type=jnp.float32))
    h = _lnorm(_gelu(h), gc[...], bec[...])
    o[...] = (lax.dot_general(h, Wc[...], (((1,), (1,)), ((), ())),
                              preferred_element_type=jnp.float32)
              + bc[...])


def _bspec(shape):
    nd = len(shape)
    return pl.BlockSpec(shape, lambda i: (i,) + (0,) * (nd - 1))


def _wspec(shape):
    nd = len(shape)
    return pl.BlockSpec(shape, lambda i: (0,) * nd)


def _tc_call(body, n_node_in, w_shapes, out_shapes):
    grid = N // BN
    in_specs = ([_bspec((BN, s)) for s in n_node_in]
                + [_wspec(s) for s in w_shapes])
    out_specs = [_bspec((BN, s)) for s in out_shapes]
    out_shape = [jax.ShapeDtypeStruct((N, s), jnp.float32) for s in out_shapes]
    if len(out_shapes) == 1:
        out_specs, out_shape = out_specs[0], out_shape[0]
    return pl.pallas_call(
        body, grid=(grid,), in_specs=in_specs,
        out_specs=out_specs, out_shape=out_shape)


_tc1 = _tc_call(_tc12_body, [D_HID, D_HID, 16, 16, D_HID],
                [(D_HID, D_HID), (1, D_HID), (D_HID, D_HID), (1, D_HID), (1, D_HID)],
                [D_HID])
_tc2 = _tc_call(_tc2_body, [D_HID, D_HID, 16, 16, D_HID],
                [(D_HID, D_HID), (1, D_HID), (D_HID, D_HID), (1, D_HID), (1, D_HID),
                 (D_OUT, D_HID)],
                [D_HID, D_OUT])
_tc3 = _tc_call(_tc3_body, [D_OUT, D_OUT, 16, 16, D_HID],
                [(D_OUT, D_HID), (1, D_OUT), (1, D_OUT), (1, D_OUT),
                 (N_CLS, D_OUT), (1, N_CLS)],
                [N_CLS])


def kernel(x, edge_index, W1l, b1l, W1r, g1, be1, W2l, b2l, W2r, g2, be2,
           Wol, bol, Wor, gc, bec, Wc, bc):
    src = edge_index[0]
    dst = edge_index[1]
    pad = E_PAD - E
    srcp = jnp.concatenate([src, jnp.zeros((pad,), jnp.int32)])
    dstp = jnp.concatenate([dst, jnp.full((pad,), N, jnp.int32)])
    z128 = jnp.zeros((CHUNK, D_HID), jnp.float32)
    z64 = jnp.zeros((CHUNK, D_OUT), jnp.float32)
    z16 = jnp.zeros((CHUNK, 16), jnp.float32)
    ones = jnp.ones((CHUNK, 16), jnp.float32)
    r = lambda v: v.reshape(1, -1)

    cntp = _cnt(dstp, z16, ones)
    part1 = _agg128(x, srcp, dstp, z128)
    c0, c1 = cntp[0], cntp[1]
    x1 = _tc1(part1[0], part1[1], c0, c1, x,
              W1l, r(b1l), W1r, r(g1), r(be1))
    part2 = _agg128(x1, srcp, dstp, z128)
    x2, y3 = _tc2(part2[0], part2[1], c0, c1, x1,
                  W2l, r(b2l), W2r, r(g2), r(be2), Wol)
    part3 = _agg64(y3, srcp, dstp, z64)
    return _tc3(part3[0], part3[1], c0, c1, x2,
                Wor, r(bol), r(gc), r(bec), Wc, r(bc))
